# baseline (speedup 1.0000x reference)
import functools

import jax
import jax.numpy as jnp
from jax import lax
from jax.experimental import pallas as pl
from jax.experimental.pallas import tpu as pltpu
from jax.experimental.pallas import tpu_sc as plsc

B, S, H = 4096, 200, 64
N = B * S
MAX_POS = 512
TYPE_VOCAB = 2
EPS = 1e-12

NC, NS, L = 2, 16, 16
NW = NC * NS
TOK_PER_W = N // NW
C = 128
N_CHUNKS = TOK_PER_W // C
G = C // L


def _sc_body(ids, pt_ids, obj_ids, word_t, combo_t, obj_t, ln_g, ln_b,
             emb_out, obj_out,
             idx0, idx1, w0, w1, p0, p1, o0, o1, u0, u1, gbuf, bbuf, grot, brot, tmp,
             s_idx0, s_idx1, s_gw0, s_gw1, s_gp0, s_gp1, s_go0, s_go1,
             s_os0, s_os1, s_es0, s_es1):
    idxb = (idx0, idx1)
    wb, pb, ob, ub = (w0, w1), (p0, p1), (o0, o1), (u0, u1)
    s_idx, s_gw, s_gp, s_go = (s_idx0, s_idx1), (s_gw0, s_gw1), (s_gp0, s_gp1), (s_go0, s_go1)
    s_os, s_es = (s_os0, s_os1), (s_es0, s_es1)

    wid = lax.axis_index("s") * NC + lax.axis_index("c")
    tok0 = wid * TOK_PER_W
    pltpu.sync_copy(ln_g, gbuf)
    pltpu.sync_copy(ln_b, bbuf)

    def idx_copies(c, b):
        base = tok0 + c * C
        return (pltpu.make_async_copy(ids.at[pl.ds(base, C)], idxb[b].at[0], s_idx[b]),
                pltpu.make_async_copy(pt_ids.at[pl.ds(base, C)], idxb[b].at[1], s_idx[b]),
                pltpu.make_async_copy(obj_ids.at[pl.ds(base, C)], idxb[b].at[2], s_idx[b]))

    def gather_copies(b):
        return (pltpu.make_async_copy(word_t.at[idxb[b].at[0]], wb[b], s_gw[b]),
                pltpu.make_async_copy(combo_t.at[idxb[b].at[1]], pb[b], s_gp[b]),
                pltpu.make_async_copy(obj_t.at[idxb[b].at[2]], ob[b], s_go[b]))

    def ow_copy(c, b):
        return pltpu.make_async_copy(ob[b], obj_out.at[pl.ds(tok0 + c * C, C)], s_os[b])

    def ew_copy(c, b):
        return pltpu.make_async_copy(ub[b], emb_out.at[pl.ds(tok0 + c * C, C)], s_es[b])

    lane = lax.iota(jnp.int32, L)
    for h in range(H):
        hv = (h + lane) & (H - 1)
        grot[pl.ds(h * L, L)] = plsc.load_gather(gbuf, [hv])
        brot[pl.ds(h * L, L)] = plsc.load_gather(bbuf, [hv])

    def compute(b):
        @plsc.parallel_loop(0, G, 1, unroll=2)
        def group(g):
            tok = g * L + lax.iota(jnp.int32, L)
            skew = lax.iota(jnp.int32, L)
            acc = [jnp.zeros((L,), jnp.float32) for _ in range(4)]
            for h in range(0, H, 2):
                hv0 = (skew + h) & (H - 1)
                hv1 = (skew + (h + 1)) & (H - 1)
                w0 = plsc.load_gather(wb[b], [tok, hv0])
                p0 = plsc.load_gather(pb[b], [tok, hv0])
                w1 = plsc.load_gather(wb[b], [tok, hv1])
                p1 = plsc.load_gather(pb[b], [tok, hv1])
                x0 = w0 + p0
                x1 = w1 + p1
                tmp[g, pl.ds(h * L, L)] = x0
                tmp[g, pl.ds((h + 1) * L, L)] = x1
                acc[0] = acc[0] + x0
                acc[1] = acc[1] + x1
                acc[2] = acc[2] + x0 * x0
                acc[3] = acc[3] + x1 * x1
            mu = (acc[0] + acc[1]) * (1.0 / H)
            var = (acc[2] + acc[3]) * (1.0 / H) - mu * mu + EPS
            i = plsc.bitcast(var, jnp.int32)
            y = plsc.bitcast(jnp.int32(0x5F3759DF) - lax.shift_right_arithmetic(i, 1),
                             jnp.float32)
            for _ in range(3):
                y = y * (1.5 - 0.5 * var * y * y)
            for h in range(0, H, 2):
                hv0 = (skew + h) & (H - 1)
                hv1 = (skew + (h + 1)) & (H - 1)
                x0 = tmp[g, pl.ds(h * L, L)]
                x1 = tmp[g, pl.ds((h + 1) * L, L)]
                o0 = (x0 - mu) * y * grot[pl.ds(h * L, L)] + brot[pl.ds(h * L, L)]
                o1 = (x1 - mu) * y * grot[pl.ds((h + 1) * L, L)] + brot[pl.ds((h + 1) * L, L)]
                plsc.store_scatter(ub[b], [tok, hv0], o0)
                plsc.store_scatter(ub[b], [tok, hv1], o1)

    for d in idx_copies(0, 0):
        d.start()
    for d in idx_copies(1, 1):
        d.start()
    for d in idx_copies(0, 0):
        d.wait()
    for d in gather_copies(0):
        d.start()

    def outer(i, carry):
        for b in (0, 1):
            c = 2 * i + b
            nb = 1 - b

            @pl.when(c + 1 < N_CHUNKS)
            def _():
                for d in idx_copies(c + 1, nb):
                    d.wait()

            @pl.when(jnp.logical_and(c >= 1, c + 1 < N_CHUNKS))
            def _():
                ow_copy(c - 1, nb).wait()
                ew_copy(c - 1, nb).wait()

            @pl.when(c + 1 < N_CHUNKS)
            def _():
                for d in gather_copies(nb):
                    d.start()

            for d in gather_copies(b):
                d.wait()

            @pl.when(c + 2 < N_CHUNKS)
            def _():
                for d in idx_copies(c + 2, b):
                    d.start()

            ow_copy(c, b).start()
            compute(b)
            ew_copy(c, b).start()
        return carry

    lax.fori_loop(0, N_CHUNKS // 2, outer, 0)

    ow_copy(N_CHUNKS - 2, 0).wait()
    ew_copy(N_CHUNKS - 2, 0).wait()
    ow_copy(N_CHUNKS - 1, 1).wait()
    ew_copy(N_CHUNKS - 1, 1).wait()


_sc_call = functools.partial(
    pl.kernel,
    out_type=(jax.ShapeDtypeStruct((N, H), jnp.float32),
              jax.ShapeDtypeStruct((N, H), jnp.float32)),
    mesh=plsc.VectorSubcoreMesh(core_axis_name="c", subcore_axis_name="s"),
    compiler_params=pltpu.CompilerParams(needs_layout_passes=False,
                                         use_tc_tiling_on_sc=False),
    scratch_types=[
        pltpu.VMEM((3, C), jnp.int32),
        pltpu.VMEM((3, C), jnp.int32),
        pltpu.VMEM((C, H), jnp.float32),
        pltpu.VMEM((C, H), jnp.float32),
        pltpu.VMEM((C, H), jnp.float32),
        pltpu.VMEM((C, H), jnp.float32),
        pltpu.VMEM((C, H), jnp.float32),
        pltpu.VMEM((C, H), jnp.float32),
        pltpu.VMEM((C, H), jnp.float32),
        pltpu.VMEM((C, H), jnp.float32),
        pltpu.VMEM((H,), jnp.float32),
        pltpu.VMEM((H,), jnp.float32),
        pltpu.VMEM((H * L,), jnp.float32),
        pltpu.VMEM((H * L,), jnp.float32),
        pltpu.VMEM((G, H * L), jnp.float32),
        pltpu.SemaphoreType.DMA,
        pltpu.SemaphoreType.DMA,
        pltpu.SemaphoreType.DMA,
        pltpu.SemaphoreType.DMA,
        pltpu.SemaphoreType.DMA,
        pltpu.SemaphoreType.DMA,
        pltpu.SemaphoreType.DMA,
        pltpu.SemaphoreType.DMA,
        pltpu.SemaphoreType.DMA,
        pltpu.SemaphoreType.DMA,
        pltpu.SemaphoreType.DMA,
        pltpu.SemaphoreType.DMA,
    ],
)(_sc_body)


def kernel(input_ids, token_type_ids, position_ids, act_txt, obj_txt,
           word_table, pos_table, type_table, obj_table, ln_gamma, ln_beta):
    del act_txt
    ids = input_ids.reshape(N)
    pt_ids = (position_ids * TYPE_VOCAB + token_type_ids).reshape(N)
    obj_ids = obj_txt.reshape(N)
    combo = (pos_table[:, None, :] + type_table[None, :, :]).reshape(
        MAX_POS * TYPE_VOCAB, H)
    emb, obj = _sc_call(ids, pt_ids, obj_ids, word_table, combo, obj_table,
                        ln_gamma, ln_beta)
    return emb.reshape(B, S, H), obj.reshape(B, S, H)


# speedup vs baseline: 1.2506x; 1.2506x over previous
import functools

import jax
import jax.numpy as jnp
from jax import lax
from jax.experimental import pallas as pl
from jax.experimental.pallas import tpu as pltpu
from jax.experimental.pallas import tpu_sc as plsc

B, S, H = 4096, 200, 64
N = B * S
MAX_POS = 512
TYPE_VOCAB = 2
EPS = 1e-12

NC, NS, L = 2, 16, 16
NW = NC * NS
TOK_PER_W = N // NW
IW = 128
C = 256
N_CHUNKS = TOK_PER_W // C
ROWS_PER_W = TOK_PER_W // IW
G = C // L


def _sc_body(ids, pt_ids, obj_ids, word_t, combo_t, obj_t, ln_g, ln_b,
             emb_out, obj_out,
             idx0, idx1, w0, w1, p0, p1, o0, o1, ubuf, gbuf, bbuf, grot, brot,
             s_idx0, s_idx1, s_gw0, s_gw1, s_gp0, s_gp1, s_go0, s_go1,
             s_os0, s_os1, s_es):
    idxb = (idx0, idx1)
    wb, pb, ob = (w0, w1), (p0, p1), (o0, o1)
    s_idx, s_gw, s_gp, s_go = (s_idx0, s_idx1), (s_gw0, s_gw1), (s_gp0, s_gp1), (s_go0, s_go1)
    s_os = (s_os0, s_os1)

    wid = lax.axis_index("s") * NC + lax.axis_index("c")
    tok0 = wid * TOK_PER_W
    row0 = wid * ROWS_PER_W
    pltpu.sync_copy(ln_g, gbuf)
    pltpu.sync_copy(ln_b, bbuf)

    lane = lax.iota(jnp.int32, L)
    for h in range(H):
        hv = (h + lane) & (H - 1)
        grot[pl.ds(h * L, L)] = plsc.load_gather(gbuf, [hv])
        brot[pl.ds(h * L, L)] = plsc.load_gather(bbuf, [hv])

    def idx_copies(c, b):
        r = row0 + c * (C // IW)
        return (pltpu.make_async_copy(ids.at[pl.ds(r, C // IW)], idxb[b].at[0], s_idx[b]),
                pltpu.make_async_copy(pt_ids.at[pl.ds(r, C // IW)], idxb[b].at[1], s_idx[b]),
                pltpu.make_async_copy(obj_ids.at[pl.ds(r, C // IW)], idxb[b].at[2], s_idx[b]))

    def gather_copies(b):
        out = []
        for j in range(C // IW):
            sl = pl.ds(j * IW, IW)
            out.append(pltpu.make_async_copy(word_t.at[idxb[b].at[0, j]], wb[b].at[sl, :], s_gw[b]))
            out.append(pltpu.make_async_copy(combo_t.at[idxb[b].at[1, j]], pb[b].at[sl, :], s_gp[b]))
            out.append(pltpu.make_async_copy(obj_t.at[idxb[b].at[2, j]], ob[b].at[sl, :], s_go[b]))
        return out

    def ow_copy(c, b):
        return pltpu.make_async_copy(ob[b], obj_out.at[pl.ds(tok0 + c * C, C)], s_os[b])

    def ew_copy(c):
        return pltpu.make_async_copy(ubuf, emb_out.at[pl.ds(tok0 + c * C, C)], s_es)

    def compute(wcur, pcur):
        @plsc.parallel_loop(0, G, 1, unroll=2)
        def group(g):
            tok = g * L + lax.iota(jnp.int32, L)
            skew = lax.iota(jnp.int32, L)
            acc = [jnp.zeros((L,), jnp.float32) for _ in range(4)]
            for h in range(0, H, 2):
                hv0 = (skew + h) & (H - 1)
                hv1 = (skew + (h + 1)) & (H - 1)
                wv0 = plsc.load_gather(wcur, [tok, hv0])
                pv0 = plsc.load_gather(pcur, [tok, hv0])
                wv1 = plsc.load_gather(wcur, [tok, hv1])
                pv1 = plsc.load_gather(pcur, [tok, hv1])
                x0 = wv0 + pv0
                x1 = wv1 + pv1
                acc[0] = acc[0] + x0
                acc[1] = acc[1] + x1
                acc[2] = acc[2] + x0 * x0
                acc[3] = acc[3] + x1 * x1
            mu = (acc[0] + acc[1]) * (1.0 / H)
            var = (acc[2] + acc[3]) * (1.0 / H) - mu * mu + EPS
            i = plsc.bitcast(var, jnp.int32)
            y = plsc.bitcast(jnp.int32(0x5F3759DF) - lax.shift_right_arithmetic(i, 1),
                             jnp.float32)
            for _ in range(3):
                y = y * (1.5 - 0.5 * var * y * y)
            for h in range(0, H, 2):
                hv0 = (skew + h) & (H - 1)
                hv1 = (skew + (h + 1)) & (H - 1)
                wv0 = plsc.load_gather(wcur, [tok, hv0])
                pv0 = plsc.load_gather(pcur, [tok, hv0])
                wv1 = plsc.load_gather(wcur, [tok, hv1])
                pv1 = plsc.load_gather(pcur, [tok, hv1])
                x0 = wv0 + pv0
                x1 = wv1 + pv1
                o0 = (x0 - mu) * y * grot[pl.ds(h * L, L)] + brot[pl.ds(h * L, L)]
                o1 = (x1 - mu) * y * grot[pl.ds((h + 1) * L, L)] + brot[pl.ds((h + 1) * L, L)]
                plsc.store_scatter(ubuf, [tok, hv0], o0)
                plsc.store_scatter(ubuf, [tok, hv1], o1)

    for d in idx_copies(0, 0):
        d.start()
    for d in idx_copies(1, 1):
        d.start()
    for d in idx_copies(0, 0):
        d.wait()
    for d in gather_copies(0):
        d.start()

    def outer(i, carry):
        for b in (0, 1):
            c = 2 * i + b
            nb = 1 - b

            @pl.when(c + 1 < N_CHUNKS)
            def _():
                for d in idx_copies(c + 1, nb):
                    d.wait()

            @pl.when(jnp.logical_and(c >= 1, c + 1 < N_CHUNKS))
            def _():
                ow_copy(c - 1, nb).wait()

            @pl.when(c + 1 < N_CHUNKS)
            def _():
                for d in gather_copies(nb):
                    d.start()

            for d in gather_copies(b):
                d.wait()

            @pl.when(c + 2 < N_CHUNKS)
            def _():
                for d in idx_copies(c + 2, b):
                    d.start()

            ow_copy(c, b).start()

            @pl.when(c >= 1)
            def _():
                ew_copy(c - 1).wait()

            compute(wb[b], pb[b])
            ew_copy(c).start()
        return carry

    lax.fori_loop(0, N_CHUNKS // 2, outer, 0)

    ow_copy(N_CHUNKS - 2, 0).wait()
    ow_copy(N_CHUNKS - 1, 1).wait()
    ew_copy(N_CHUNKS - 1).wait()


_sc_call = functools.partial(
    pl.kernel,
    out_type=(jax.ShapeDtypeStruct((N, H), jnp.float32),
              jax.ShapeDtypeStruct((N, H), jnp.float32)),
    mesh=plsc.VectorSubcoreMesh(core_axis_name="c", subcore_axis_name="s"),
    compiler_params=pltpu.CompilerParams(needs_layout_passes=False,
                                         use_tc_tiling_on_sc=False),
    scratch_types=[
        pltpu.VMEM((3, C // IW, IW), jnp.int32),
        pltpu.VMEM((3, C // IW, IW), jnp.int32),
        pltpu.VMEM((C, H), jnp.float32),
        pltpu.VMEM((C, H), jnp.float32),
        pltpu.VMEM((C, H), jnp.float32),
        pltpu.VMEM((C, H), jnp.float32),
        pltpu.VMEM((C, H), jnp.float32),
        pltpu.VMEM((C, H), jnp.float32),
        pltpu.VMEM((C, H), jnp.float32),
        pltpu.VMEM((H,), jnp.float32),
        pltpu.VMEM((H,), jnp.float32),
        pltpu.VMEM((H * L,), jnp.float32),
        pltpu.VMEM((H * L,), jnp.float32),
        pltpu.SemaphoreType.DMA,
        pltpu.SemaphoreType.DMA,
        pltpu.SemaphoreType.DMA,
        pltpu.SemaphoreType.DMA,
        pltpu.SemaphoreType.DMA,
        pltpu.SemaphoreType.DMA,
        pltpu.SemaphoreType.DMA,
        pltpu.SemaphoreType.DMA,
        pltpu.SemaphoreType.DMA,
        pltpu.SemaphoreType.DMA,
        pltpu.SemaphoreType.DMA,
    ],
)(_sc_body)


def kernel(input_ids, token_type_ids, position_ids, act_txt, obj_txt,
           word_table, pos_table, type_table, obj_table, ln_gamma, ln_beta):
    del act_txt
    ids = input_ids.reshape(N // IW, IW)
    pt_ids = (position_ids * TYPE_VOCAB + token_type_ids).reshape(N // IW, IW)
    obj_ids = obj_txt.reshape(N // IW, IW)
    combo = (pos_table[:, None, :] + type_table[None, :, :]).reshape(
        MAX_POS * TYPE_VOCAB, H)
    emb, obj = _sc_call(ids, pt_ids, obj_ids, word_table, combo, obj_table,
                        ln_gamma, ln_beta)
    return emb.reshape(B, S, H), obj.reshape(B, S, H)


# speedup vs baseline: 1.4215x; 1.1367x over previous
import functools

import jax
import jax.numpy as jnp
from jax import lax
from jax.experimental import pallas as pl
from jax.experimental.pallas import tpu as pltpu
from jax.experimental.pallas import tpu_sc as plsc

B, S, H = 4096, 200, 64
N = B * S
MAX_POS = 512
TYPE_VOCAB = 2
EPS = 1e-12

NC, NS, L = 2, 16, 16
NW = NC * NS
TOK_PER_W = N // NW
IW = 128
C = 256
N_CHUNKS = TOK_PER_W // C
ROWS_PER_W = TOK_PER_W // IW
G = C // L


def _sc_body(ids, pt_ids, obj_ids, word_t, combo_t, obj_t, ln_g, ln_b,
             emb_out, obj_out,
             idx0, idx1, w0, w1, p0, p1, o0, o1, ubuf, gbuf, bbuf, grot, brot,
             combo_s, obj_s,
             s_idx0, s_idx1, s_gw0, s_gw1, s_gp0, s_gp1, s_go0, s_go1,
             s_os0, s_os1, s_es):
    idxb = (idx0, idx1)
    wb, pb, ob = (w0, w1), (p0, p1), (o0, o1)
    s_idx, s_gw, s_gp, s_go = (s_idx0, s_idx1), (s_gw0, s_gw1), (s_gp0, s_gp1), (s_go0, s_go1)
    s_os = (s_os0, s_os1)

    wid = lax.axis_index("s") * NC + lax.axis_index("c")
    tok0 = wid * TOK_PER_W
    row0 = wid * ROWS_PER_W
    pltpu.sync_copy(ln_g, gbuf)
    pltpu.sync_copy(ln_b, bbuf)

    @pl.when(lax.axis_index("s") == 0)
    def _():
        pltpu.sync_copy(combo_t, combo_s)
        pltpu.sync_copy(obj_t, obj_s)
    plsc.subcore_barrier()

    lane = lax.iota(jnp.int32, L)
    for h in range(H):
        hv = (h + lane) & (H - 1)
        grot[pl.ds(h * L, L)] = plsc.load_gather(gbuf, [hv])
        brot[pl.ds(h * L, L)] = plsc.load_gather(bbuf, [hv])

    def idx_copies(c, b):
        r = row0 + c * (C // IW)
        return (pltpu.make_async_copy(ids.at[pl.ds(r, C // IW)], idxb[b].at[0], s_idx[b]),
                pltpu.make_async_copy(pt_ids.at[pl.ds(r, C // IW)], idxb[b].at[1], s_idx[b]),
                pltpu.make_async_copy(obj_ids.at[pl.ds(r, C // IW)], idxb[b].at[2], s_idx[b]))

    def gather_copies(b):
        out = []
        for j in range(C // IW):
            sl = pl.ds(j * IW, IW)
            out.append(pltpu.make_async_copy(word_t.at[idxb[b].at[0, j]], wb[b].at[sl, :], s_gw[b]))
            out.append(pltpu.make_async_copy(combo_s.at[idxb[b].at[1, j]], pb[b].at[sl, :], s_gp[b]))
            out.append(pltpu.make_async_copy(obj_s.at[idxb[b].at[2, j]], ob[b].at[sl, :], s_go[b]))
        return out

    def ow_copy(c, b):
        return pltpu.make_async_copy(ob[b], obj_out.at[pl.ds(tok0 + c * C, C)], s_os[b])

    def ew_copy(c):
        return pltpu.make_async_copy(ubuf, emb_out.at[pl.ds(tok0 + c * C, C)], s_es)

    def compute(wcur, pcur):
        @plsc.parallel_loop(0, G, 1, unroll=2)
        def group(g):
            tok = g * L + lax.iota(jnp.int32, L)
            skew = lax.iota(jnp.int32, L)
            acc = [jnp.zeros((L,), jnp.float32) for _ in range(4)]
            for h in range(0, H, 2):
                hv0 = (skew + h) & (H - 1)
                hv1 = (skew + (h + 1)) & (H - 1)
                wv0 = plsc.load_gather(wcur, [tok, hv0])
                pv0 = plsc.load_gather(pcur, [tok, hv0])
                wv1 = plsc.load_gather(wcur, [tok, hv1])
                pv1 = plsc.load_gather(pcur, [tok, hv1])
                x0 = wv0 + pv0
                x1 = wv1 + pv1
                acc[0] = acc[0] + x0
                acc[1] = acc[1] + x1
                acc[2] = acc[2] + x0 * x0
                acc[3] = acc[3] + x1 * x1
            mu = (acc[0] + acc[1]) * (1.0 / H)
            var = (acc[2] + acc[3]) * (1.0 / H) - mu * mu + EPS
            i = plsc.bitcast(var, jnp.int32)
            y = plsc.bitcast(jnp.int32(0x5F3759DF) - lax.shift_right_arithmetic(i, 1),
                             jnp.float32)
            for _ in range(3):
                y = y * (1.5 - 0.5 * var * y * y)
            for h in range(0, H, 2):
                hv0 = (skew + h) & (H - 1)
                hv1 = (skew + (h + 1)) & (H - 1)
                wv0 = plsc.load_gather(wcur, [tok, hv0])
                pv0 = plsc.load_gather(pcur, [tok, hv0])
                wv1 = plsc.load_gather(wcur, [tok, hv1])
                pv1 = plsc.load_gather(pcur, [tok, hv1])
                x0 = wv0 + pv0
                x1 = wv1 + pv1
                o0 = (x0 - mu) * y * grot[pl.ds(h * L, L)] + brot[pl.ds(h * L, L)]
                o1 = (x1 - mu) * y * grot[pl.ds((h + 1) * L, L)] + brot[pl.ds((h + 1) * L, L)]
                plsc.store_scatter(ubuf, [tok, hv0], o0)
                plsc.store_scatter(ubuf, [tok, hv1], o1)

    for d in idx_copies(0, 0):
        d.start()
    for d in idx_copies(1, 1):
        d.start()
    for d in idx_copies(0, 0):
        d.wait()
    for d in gather_copies(0):
        d.start()

    def outer(i, carry):
        for b in (0, 1):
            c = 2 * i + b
            nb = 1 - b

            @pl.when(c + 1 < N_CHUNKS)
            def _():
                for d in idx_copies(c + 1, nb):
                    d.wait()

            @pl.when(jnp.logical_and(c >= 1, c + 1 < N_CHUNKS))
            def _():
                ow_copy(c - 1, nb).wait()

            @pl.when(c + 1 < N_CHUNKS)
            def _():
                for d in gather_copies(nb):
                    d.start()

            for d in gather_copies(b):
                d.wait()

            @pl.when(c + 2 < N_CHUNKS)
            def _():
                for d in idx_copies(c + 2, b):
                    d.start()

            ow_copy(c, b).start()

            @pl.when(c >= 1)
            def _():
                ew_copy(c - 1).wait()

            compute(wb[b], pb[b])
            ew_copy(c).start()
        return carry

    lax.fori_loop(0, N_CHUNKS // 2, outer, 0)

    ow_copy(N_CHUNKS - 2, 0).wait()
    ow_copy(N_CHUNKS - 1, 1).wait()
    ew_copy(N_CHUNKS - 1).wait()


_sc_call = functools.partial(
    pl.kernel,
    out_type=(jax.ShapeDtypeStruct((N, H), jnp.float32),
              jax.ShapeDtypeStruct((N, H), jnp.float32)),
    mesh=plsc.VectorSubcoreMesh(core_axis_name="c", subcore_axis_name="s"),
    compiler_params=pltpu.CompilerParams(needs_layout_passes=False,
                                         use_tc_tiling_on_sc=False),
    scratch_types=[
        pltpu.VMEM((3, C // IW, IW), jnp.int32),
        pltpu.VMEM((3, C // IW, IW), jnp.int32),
        pltpu.VMEM((C, H), jnp.float32),
        pltpu.VMEM((C, H), jnp.float32),
        pltpu.VMEM((C, H), jnp.float32),
        pltpu.VMEM((C, H), jnp.float32),
        pltpu.VMEM((C, H), jnp.float32),
        pltpu.VMEM((C, H), jnp.float32),
        pltpu.VMEM((C, H), jnp.float32),
        pltpu.VMEM((H,), jnp.float32),
        pltpu.VMEM((H,), jnp.float32),
        pltpu.VMEM((H * L,), jnp.float32),
        pltpu.VMEM((H * L,), jnp.float32),
        pltpu.VMEM_SHARED((MAX_POS * TYPE_VOCAB, H), jnp.float32),
        pltpu.VMEM_SHARED((1000, H), jnp.float32),
        pltpu.SemaphoreType.DMA,
        pltpu.SemaphoreType.DMA,
        pltpu.SemaphoreType.DMA,
        pltpu.SemaphoreType.DMA,
        pltpu.SemaphoreType.DMA,
        pltpu.SemaphoreType.DMA,
        pltpu.SemaphoreType.DMA,
        pltpu.SemaphoreType.DMA,
        pltpu.SemaphoreType.DMA,
        pltpu.SemaphoreType.DMA,
        pltpu.SemaphoreType.DMA,
    ],
)(_sc_body)


def kernel(input_ids, token_type_ids, position_ids, act_txt, obj_txt,
           word_table, pos_table, type_table, obj_table, ln_gamma, ln_beta):
    del act_txt
    ids = input_ids.reshape(N // IW, IW)
    pt_ids = (position_ids * TYPE_VOCAB + token_type_ids).reshape(N // IW, IW)
    obj_ids = obj_txt.reshape(N // IW, IW)
    combo = (pos_table[:, None, :] + type_table[None, :, :]).reshape(
        MAX_POS * TYPE_VOCAB, H)
    emb, obj = _sc_call(ids, pt_ids, obj_ids, word_table, combo, obj_table,
                        ln_gamma, ln_beta)
    return emb.reshape(B, S, H), obj.reshape(B, S, H)


# speedup vs baseline: 2.0115x; 1.4150x over previous
import functools

import jax
import jax.numpy as jnp
from jax import lax
from jax.experimental import pallas as pl
from jax.experimental.pallas import tpu as pltpu
from jax.experimental.pallas import tpu_sc as plsc

B, S, H = 4096, 200, 64
N = B * S
MAX_POS = 512
TYPE_VOCAB = 2
EPS = 1e-12

NC, NS, L = 2, 16, 16
NW = NC * NS
TOK_PER_W = N // NW
IW = 128
C = 256
N_CHUNKS = TOK_PER_W // C
ROWS_PER_W = TOK_PER_W // IW
G = C // L


def _sc_body(ids, pt_ids, obj_ids, word_t, combo_t, obj_t, ln_g, ln_b,
             emb_out, obj_out,
             idx0, idx1, w0, w1, p0, p1, o0, o1, ubuf, gbuf, bbuf, grot, brot,
             combo_s, obj_s,
             s_idx0, s_idx1, s_gw0, s_gw1, s_gp0, s_gp1, s_go0, s_go1,
             s_os0, s_os1, s_es):
    idxb = (idx0, idx1)
    wb, pb, ob = (w0, w1), (p0, p1), (o0, o1)
    s_idx, s_gw, s_gp, s_go = (s_idx0, s_idx1), (s_gw0, s_gw1), (s_gp0, s_gp1), (s_go0, s_go1)
    s_os = (s_os0, s_os1)

    wid = lax.axis_index("s") * NC + lax.axis_index("c")
    tok0 = wid * TOK_PER_W
    row0 = wid * ROWS_PER_W
    pltpu.sync_copy(ln_g, gbuf)
    pltpu.sync_copy(ln_b, bbuf)

    @pl.when(lax.axis_index("s") == 0)
    def _():
        pltpu.sync_copy(combo_t, combo_s)
        pltpu.sync_copy(obj_t, obj_s)
    plsc.subcore_barrier()

    lane = lax.iota(jnp.int32, L)
    for h in range(H):
        hv = (h + lane) & (H - 1)
        grot[pl.ds(h * L, L)] = plsc.load_gather(gbuf, [hv])
        brot[pl.ds(h * L, L)] = plsc.load_gather(bbuf, [hv])

    def idx_copies(c, b):
        r = row0 + c * (C // IW)
        return (pltpu.make_async_copy(ids.at[pl.ds(r, C // IW)], idxb[b].at[0], s_idx[b]),
                pltpu.make_async_copy(pt_ids.at[pl.ds(r, C // IW)], idxb[b].at[1], s_idx[b]),
                pltpu.make_async_copy(obj_ids.at[pl.ds(r, C // IW)], idxb[b].at[2], s_idx[b]))

    def gather_copies(b):
        out = []
        for j in range(C // IW):
            sl = pl.ds(j * IW, IW)
            out.append(pltpu.make_async_copy(word_t.at[idxb[b].at[0, j]], wb[b].at[sl, :], s_gw[b]))
            out.append(pltpu.make_async_copy(combo_s.at[idxb[b].at[1, j]], pb[b].at[sl, :], s_gp[b]))
            out.append(pltpu.make_async_copy(obj_s.at[idxb[b].at[2, j]], ob[b].at[sl, :], s_go[b]))
        return out

    def ow_copy(c, b):
        return pltpu.make_async_copy(ob[b], obj_out.at[pl.ds(tok0 + c * C, C)], s_os[b])

    def ew_copy(c):
        return pltpu.make_async_copy(ubuf, emb_out.at[pl.ds(tok0 + c * C, C)], s_es)

    def compute(wcur, pcur):
        @plsc.parallel_loop(0, G, 1)
        def group(g):
            tok = g * L + lax.iota(jnp.int32, L)
            skew = lax.iota(jnp.int32, L)
            zero = jnp.zeros((L,), jnp.float32)

            @plsc.parallel_loop(0, H, 4, unroll=2, carry=(zero, zero, zero, zero))
            def pass1(h, acc):
                a0, a1, q0, q1 = acc
                hv0 = (skew + h) & (H - 1)
                hv1 = (skew + h + 1) & (H - 1)
                hv2 = (skew + h + 2) & (H - 1)
                hv3 = (skew + h + 3) & (H - 1)
                x0 = plsc.load_gather(wcur, [tok, hv0]) + plsc.load_gather(pcur, [tok, hv0])
                x1 = plsc.load_gather(wcur, [tok, hv1]) + plsc.load_gather(pcur, [tok, hv1])
                x2 = plsc.load_gather(wcur, [tok, hv2]) + plsc.load_gather(pcur, [tok, hv2])
                x3 = plsc.load_gather(wcur, [tok, hv3]) + plsc.load_gather(pcur, [tok, hv3])
                return (a0 + (x0 + x2), a1 + (x1 + x3),
                        q0 + (x0 * x0 + x2 * x2), q1 + (x1 * x1 + x3 * x3))

            a0, a1, q0, q1 = pass1
            mu = (a0 + a1) * (1.0 / H)
            var = (q0 + q1) * (1.0 / H) - mu * mu + EPS
            i = plsc.bitcast(var, jnp.int32)
            y = plsc.bitcast(jnp.int32(0x5F3759DF) - lax.shift_right_arithmetic(i, 1),
                             jnp.float32)
            for _ in range(3):
                y = y * (1.5 - 0.5 * var * y * y)

            @plsc.parallel_loop(0, H, 2, unroll=4)
            def pass2(h):
                hv0 = (skew + h) & (H - 1)
                hv1 = (skew + h + 1) & (H - 1)
                x0 = plsc.load_gather(wcur, [tok, hv0]) + plsc.load_gather(pcur, [tok, hv0])
                x1 = plsc.load_gather(wcur, [tok, hv1]) + plsc.load_gather(pcur, [tok, hv1])
                o0 = (x0 - mu) * y * grot[pl.ds(h * L, L)] + brot[pl.ds(h * L, L)]
                o1 = (x1 - mu) * y * grot[pl.ds(h * L + L, L)] + brot[pl.ds(h * L + L, L)]
                plsc.store_scatter(ubuf, [tok, hv0], o0)
                plsc.store_scatter(ubuf, [tok, hv1], o1)

    for d in idx_copies(0, 0):
        d.start()
    for d in idx_copies(1, 1):
        d.start()
    for d in idx_copies(0, 0):
        d.wait()
    for d in gather_copies(0):
        d.start()

    def outer(i, carry):
        for b in (0, 1):
            c = 2 * i + b
            nb = 1 - b

            @pl.when(c + 1 < N_CHUNKS)
            def _():
                for d in idx_copies(c + 1, nb):
                    d.wait()

            @pl.when(jnp.logical_and(c >= 1, c + 1 < N_CHUNKS))
            def _():
                ow_copy(c - 1, nb).wait()

            @pl.when(c + 1 < N_CHUNKS)
            def _():
                for d in gather_copies(nb):
                    d.start()

            for d in gather_copies(b):
                d.wait()

            @pl.when(c + 2 < N_CHUNKS)
            def _():
                for d in idx_copies(c + 2, b):
                    d.start()

            ow_copy(c, b).start()

            @pl.when(c >= 1)
            def _():
                ew_copy(c - 1).wait()

            compute(wb[b], pb[b])
            ew_copy(c).start()
        return carry

    lax.fori_loop(0, N_CHUNKS // 2, outer, 0)

    ow_copy(N_CHUNKS - 2, 0).wait()
    ow_copy(N_CHUNKS - 1, 1).wait()
    ew_copy(N_CHUNKS - 1).wait()


_sc_call = functools.partial(
    pl.kernel,
    out_type=(jax.ShapeDtypeStruct((N, H), jnp.float32),
              jax.ShapeDtypeStruct((N, H), jnp.float32)),
    mesh=plsc.VectorSubcoreMesh(core_axis_name="c", subcore_axis_name="s"),
    compiler_params=pltpu.CompilerParams(needs_layout_passes=False,
                                         use_tc_tiling_on_sc=False),
    scratch_types=[
        pltpu.VMEM((3, C // IW, IW), jnp.int32),
        pltpu.VMEM((3, C // IW, IW), jnp.int32),
        pltpu.VMEM((C, H), jnp.float32),
        pltpu.VMEM((C, H), jnp.float32),
        pltpu.VMEM((C, H), jnp.float32),
        pltpu.VMEM((C, H), jnp.float32),
        pltpu.VMEM((C, H), jnp.float32),
        pltpu.VMEM((C, H), jnp.float32),
        pltpu.VMEM((C, H), jnp.float32),
        pltpu.VMEM((H,), jnp.float32),
        pltpu.VMEM((H,), jnp.float32),
        pltpu.VMEM((H * L,), jnp.float32),
        pltpu.VMEM((H * L,), jnp.float32),
        pltpu.VMEM_SHARED((MAX_POS * TYPE_VOCAB, H), jnp.float32),
        pltpu.VMEM_SHARED((1000, H), jnp.float32),
        pltpu.SemaphoreType.DMA,
        pltpu.SemaphoreType.DMA,
        pltpu.SemaphoreType.DMA,
        pltpu.SemaphoreType.DMA,
        pltpu.SemaphoreType.DMA,
        pltpu.SemaphoreType.DMA,
        pltpu.SemaphoreType.DMA,
        pltpu.SemaphoreType.DMA,
        pltpu.SemaphoreType.DMA,
        pltpu.SemaphoreType.DMA,
        pltpu.SemaphoreType.DMA,
    ],
)(_sc_body)


def kernel(input_ids, token_type_ids, position_ids, act_txt, obj_txt,
           word_table, pos_table, type_table, obj_table, ln_gamma, ln_beta):
    del act_txt
    ids = input_ids.reshape(N // IW, IW)
    pt_ids = (position_ids * TYPE_VOCAB + token_type_ids).reshape(N // IW, IW)
    obj_ids = obj_txt.reshape(N // IW, IW)
    combo = (pos_table[:, None, :] + type_table[None, :, :]).reshape(
        MAX_POS * TYPE_VOCAB, H)
    emb, obj = _sc_call(ids, pt_ids, obj_ids, word_table, combo, obj_table,
                        ln_gamma, ln_beta)
    return emb.reshape(B, S, H), obj.reshape(B, S, H)
